# Initial kernel scaffold; baseline (speedup 1.0000x reference)
import jax
import jax.numpy as jnp
from jax.experimental import pallas as pl


def kernel(x, edge_index, W1, b1, W2, b2, gamma, beta):
    raise NotImplementedError("write your pallas kernel here")


# speedup vs baseline: 4.6382x; 4.6382x over previous
import functools

import jax
import jax.numpy as jnp
from jax import lax
from jax.experimental import pallas as pl
from jax.experimental.pallas import tpu as pltpu
from jax.experimental.pallas import tpu_sc as plsc

N_NODES = 10000
N_EDGES = 320000
D = 128

NC = 2
NS = 16
NW = NC * NS
E_PER_W = N_EDGES // NW
BATCH = 80
N_BATCH = E_PER_W // BATCH
N_PAD = 10240
ROWS_PER_TILE = N_PAD // NS


def _sc_aggregate(x, src3, dst3, zeros_pad):
    mesh = plsc.VectorSubcoreMesh(core_axis_name="c", subcore_axis_name="s")

    @functools.partial(
        pl.kernel,
        mesh=mesh,
        out_type=jax.ShapeDtypeStruct((NC, N_PAD, D), jnp.float32),
        scratch_types=[
            pltpu.VMEM((BATCH,), jnp.int32),
            pltpu.VMEM((BATCH,), jnp.int32),
            pltpu.VMEM((BATCH, D), jnp.float32),
            pltpu.VMEM_SHARED((N_PAD, D), jnp.float32),
            pltpu.SemaphoreType.DMA,
        ],
    )
    def agg_kernel(x_hbm, src_hbm, dst_hbm, zero_hbm, out_hbm,
                   src_v, dst_v, rows_v, acc_sh, sem):
        c = lax.axis_index("c")
        s = lax.axis_index("s")
        wid = c * NS + s

        row0 = s * ROWS_PER_TILE
        pltpu.sync_copy(zero_hbm.at[pl.ds(row0, ROWS_PER_TILE)],
                        acc_sh.at[pl.ds(row0, ROWS_PER_TILE)])
        plsc.subcore_barrier()

        def body(i, carry):
            pltpu.sync_copy(src_hbm.at[wid, i], src_v)
            pltpu.sync_copy(dst_hbm.at[wid, i], dst_v)
            pltpu.async_copy(x_hbm.at[src_v], rows_v, sem).wait()
            pltpu.sync_copy(rows_v, acc_sh.at[dst_v], add=True)
            return carry

        lax.fori_loop(0, N_BATCH, body, 0)
        plsc.subcore_barrier()

        pltpu.sync_copy(acc_sh.at[pl.ds(row0, ROWS_PER_TILE)],
                        out_hbm.at[c, pl.ds(row0, ROWS_PER_TILE)])

    return agg_kernel(x, src3, dst3, zeros_pad)


def _tc_mlp_bn(x, partials, W1, b1, W2, b2, gamma, beta):
    def body(x_ref, p_ref, w1_ref, b1_ref, w2_ref, b2_ref, g_ref, bt_ref, o_ref):
        h = x_ref[...] + p_ref[0, :N_NODES, :] + p_ref[1, :N_NODES, :]
        h = lax.dot_general(h, w1_ref[...], (((1,), (1,)), ((), ())),
                            preferred_element_type=jnp.float32,
                            precision=lax.Precision.HIGHEST)
        h = jnp.maximum(h + b1_ref[...], 0.0)
        h = lax.dot_general(h, w2_ref[...], (((1,), (1,)), ((), ())),
                            preferred_element_type=jnp.float32,
                            precision=lax.Precision.HIGHEST)
        h = jnp.maximum(h + b2_ref[...], 0.0)
        mean = jnp.mean(h, axis=0, keepdims=True)
        var = jnp.mean(h * h, axis=0, keepdims=True) - mean * mean
        o_ref[...] = (h - mean) * lax.rsqrt(var + 1e-5) * g_ref[...] + bt_ref[...]

    return pl.pallas_call(
        body,
        out_shape=jax.ShapeDtypeStruct((N_NODES, D), jnp.float32),
    )(x, partials, W1, b1, W2, b2, gamma, beta)


def kernel(x, edge_index, W1, b1, W2, b2, gamma, beta):
    src = edge_index[0].astype(jnp.int32).reshape(NW, N_BATCH, BATCH)
    dst = edge_index[1].astype(jnp.int32).reshape(NW, N_BATCH, BATCH)
    zeros_pad = jnp.zeros((N_PAD, D), jnp.float32)
    partials = _sc_aggregate(x, src, dst, zeros_pad)
    return _tc_mlp_bn(x, partials,
                      W1, b1.reshape(1, D), W2, b2.reshape(1, D),
                      gamma.reshape(1, D), beta.reshape(1, D))
